# baseline (speedup 1.0000x reference)
import dataclasses
import functools

import jax
import jax.numpy as jnp
from jax import lax
from jax.experimental import pallas as pl
from jax.experimental.pallas import tpu as pltpu
from jax.experimental.pallas import tpu_sc as plsc

N_NODES = 10000
N_EDGES = 320000
D_FEAT = 128
HID = 128
N_CLASSES = 16

NC = 2
NS = 16
NW = NC * NS
LANES = 16
CH = 128
K_CHUNKS = -(-(-(-N_EDGES // (NW * CH))) // 4) * 4
KH = K_CHUNKS // 2
E_PER_TILE = K_CHUNKS * CH
E_PAD = NW * E_PER_TILE
ROWS_MAIN = (N_NODES // NS) // 8 * 8
ROWS_TAIL = N_NODES - ROWS_MAIN * NS


def _seg_sum_sc(y, src, dst, w, d):
    mesh = plsc.VectorSubcoreMesh(core_axis_name="c", subcore_axis_name="s")
    cp = pltpu.CompilerParams()
    if "needs_layout_passes" in pltpu.CompilerParams.__dataclass_fields__:
        cp = dataclasses.replace(cp, needs_layout_passes=False)
    if d < 128:
        cp = dataclasses.replace(cp, use_tc_tiling_on_sc=False)

    @functools.partial(
        pl.kernel,
        mesh=mesh,
        compiler_params=cp,
        out_type=jax.ShapeDtypeStruct((NC, N_NODES, d), jnp.float32),
        scratch_types=[
            pltpu.VMEM((KH, CH), jnp.int32),
            pltpu.VMEM((KH, CH), jnp.int32),
            pltpu.VMEM((KH, CH), jnp.float32),
            pltpu.VMEM((CH, d), jnp.float32),
            pltpu.VMEM((CH, d), jnp.float32),
            pltpu.VMEM_SHARED((N_NODES, d), jnp.float32),
            pltpu.SemaphoreType.DMA,
            pltpu.SemaphoreType.DMA,
            pltpu.SemaphoreType.DMA,
            pltpu.SemaphoreType.DMA,
        ],
    )
    def seg_kernel(y_hbm, src_hbm, dst_hbm, w_hbm, out_hbm,
                   src_v, dst_v, w_v, rows0_v, rows1_v, acc_sh,
                   gs0, gs1, ss0, ss1):
        rows_v = rows0_v
        c = lax.axis_index("c")
        s = lax.axis_index("s")
        wid = c * NS + s

        @pl.loop(0, CH)
        def _(r):
            for j in range(d // LANES):
                rows_v[r, pl.ds(j * LANES, LANES)] = jnp.zeros(
                    (LANES,), jnp.float32)

        base = s * ROWS_MAIN
        for off in range(0, ROWS_MAIN, CH):
            n = min(CH, ROWS_MAIN - off)
            pltpu.sync_copy(rows_v.at[pl.ds(0, n)],
                            acc_sh.at[pl.ds(base + off, n)])

        @pl.when(s == NS - 1)
        def _():
            pltpu.sync_copy(rows_v.at[pl.ds(0, ROWS_TAIL)],
                            acc_sh.at[pl.ds(NS * ROWS_MAIN, ROWS_TAIL)])

        plsc.subcore_barrier()

        def scale_rows(buf, k):
            @pl.loop(0, CH)
            def _(e):
                wspl = plsc.load_gather(
                    w_v,
                    [jnp.full((LANES,), k, jnp.int32),
                     jnp.full((LANES,), e, jnp.int32)])
                for j in range(d // LANES):
                    sl = pl.ds(j * LANES, LANES)
                    buf[e, sl] = buf[e, sl] * wspl

        for h in range(2):
            h0 = h * KH
            pltpu.sync_copy(src_hbm.at[wid, pl.ds(h0, KH)], src_v)
            pltpu.sync_copy(dst_hbm.at[wid, pl.ds(h0, KH)], dst_v)
            pltpu.sync_copy(w_hbm.at[wid, pl.ds(h0, KH)], w_v)

            pltpu.async_copy(y_hbm.at[src_v.at[0]], rows0_v, gs0)
            pltpu.async_copy(y_hbm.at[src_v.at[1]], rows1_v, gs1)

            @pl.loop(0, KH, step=2)
            def _(k):
                bufs = ((rows0_v, gs0, ss0), (rows1_v, gs1, ss1))
                for b, (buf, gs, ss) in enumerate(bufs):
                    kb = k + b
                    pltpu.make_async_copy(
                        y_hbm.at[src_v.at[kb]], buf, gs).wait()
                    scale_rows(buf, kb)
                    pltpu.async_copy(buf, acc_sh.at[dst_v.at[kb]], ss,
                                     add=True)
                for b, (buf, gs, ss) in enumerate(bufs):
                    kb = k + b
                    pltpu.make_async_copy(
                        buf, acc_sh.at[dst_v.at[kb]], ss).wait()

                    @pl.when(kb + 2 < KH)
                    def _():
                        pltpu.async_copy(y_hbm.at[src_v.at[kb + 2]], buf, gs)

        plsc.subcore_barrier()

        pltpu.sync_copy(
            acc_sh.at[pl.ds(base, ROWS_MAIN)],
            out_hbm.at[c, pl.ds(base, ROWS_MAIN)])

        @pl.when(s == NS - 1)
        def _():
            pltpu.sync_copy(
                acc_sh.at[pl.ds(NS * ROWS_MAIN, ROWS_TAIL)],
                out_hbm.at[c, pl.ds(NS * ROWS_MAIN, ROWS_TAIL)])

    return seg_kernel(y, src, dst, w)


def _dot(a, b):
    return lax.dot_general(a, b, (((1,), (0,)), ((), ())),
                           precision=lax.Precision.HIGHEST,
                           preferred_element_type=jnp.float32)


_BR = 2000


def _proj1(x, w_rel, w_root):
    def body(x_ref, wa_ref, wb_ref, y_ref, r_ref):
        xv = x_ref[...]
        y_ref[...] = _dot(xv, wa_ref[...])
        r_ref[...] = _dot(xv, wb_ref[...])

    return pl.pallas_call(
        body,
        grid=(N_NODES // _BR,),
        in_specs=[
            pl.BlockSpec((_BR, D_FEAT), lambda i: (i, 0)),
            pl.BlockSpec((D_FEAT, HID), lambda i: (0, 0)),
            pl.BlockSpec((D_FEAT, HID), lambda i: (0, 0)),
        ],
        out_specs=[
            pl.BlockSpec((_BR, HID), lambda i: (i, 0)),
            pl.BlockSpec((_BR, HID), lambda i: (i, 0)),
        ],
        out_shape=[
            jax.ShapeDtypeStruct((N_NODES, HID), jnp.float32),
            jax.ShapeDtypeStruct((N_NODES, HID), jnp.float32),
        ],
    )(x, w_rel, w_root)


def _layer2_proj(agg_p, r1, b1, w2_rel, w2_root, b2):
    def body(agg_ref, r1_ref, b1_ref, wa_ref, wb_ref, b2_ref, y2_ref, r2_ref):
        h = agg_ref[0] + agg_ref[1] + r1_ref[...] + b1_ref[...]
        h = jnp.maximum(h, 0.0)
        y2_ref[...] = _dot(h, wa_ref[...])
        r2_ref[...] = _dot(h, wb_ref[...]) + b2_ref[...]

    return pl.pallas_call(
        body,
        grid=(N_NODES // _BR,),
        in_specs=[
            pl.BlockSpec((NC, _BR, HID), lambda i: (0, i, 0)),
            pl.BlockSpec((_BR, HID), lambda i: (i, 0)),
            pl.BlockSpec((1, HID), lambda i: (0, 0)),
            pl.BlockSpec((HID, N_CLASSES), lambda i: (0, 0)),
            pl.BlockSpec((HID, N_CLASSES), lambda i: (0, 0)),
            pl.BlockSpec((1, N_CLASSES), lambda i: (0, 0)),
        ],
        out_specs=[
            pl.BlockSpec((_BR, N_CLASSES), lambda i: (i, 0)),
            pl.BlockSpec((_BR, N_CLASSES), lambda i: (i, 0)),
        ],
        out_shape=[
            jax.ShapeDtypeStruct((N_NODES, N_CLASSES), jnp.float32),
            jax.ShapeDtypeStruct((N_NODES, N_CLASSES), jnp.float32),
        ],
    )(agg_p, r1, b1, w2_rel, w2_root, b2)


def _final_sum(agg_p, r2b2):
    def body(agg_ref, r_ref, o_ref):
        o_ref[...] = agg_ref[0] + agg_ref[1] + r_ref[...]

    return pl.pallas_call(
        body,
        grid=(N_NODES // _BR,),
        in_specs=[
            pl.BlockSpec((NC, _BR, N_CLASSES), lambda i: (0, i, 0)),
            pl.BlockSpec((_BR, N_CLASSES), lambda i: (i, 0)),
        ],
        out_specs=pl.BlockSpec((_BR, N_CLASSES), lambda i: (i, 0)),
        out_shape=jax.ShapeDtypeStruct((N_NODES, N_CLASSES), jnp.float32),
    )(agg_p, r2b2)


def kernel(x, edge_index, edge_attr, W1_rel, b1_rel, W1_root,
           W2_rel, b2_rel, W2_root):
    src = edge_index[0].astype(jnp.int32)
    dst = edge_index[1].astype(jnp.int32)
    pad = E_PAD - N_EDGES
    src = jnp.pad(src, (0, pad)).reshape(NW, K_CHUNKS, CH)
    dst = jnp.pad(dst, (0, pad)).reshape(NW, K_CHUNKS, CH)
    w = jnp.pad(edge_attr, (0, pad)).reshape(NW, K_CHUNKS, CH)

    y1, r1 = _proj1(x, W1_rel, W1_root)
    agg1 = _seg_sum_sc(y1, src, dst, w, HID)
    y2, r2b2 = _layer2_proj(agg1, r1, b1_rel.reshape(1, HID),
                            W2_rel, W2_root, b2_rel.reshape(1, N_CLASSES))
    agg2 = _seg_sum_sc(y2, src, dst, w, N_CLASSES)
    return _final_sum(agg2, r2b2)


# speedup vs baseline: 1.4102x; 1.4102x over previous
import dataclasses
import functools

import jax
import jax.numpy as jnp
import numpy as np
from jax import lax
from jax.experimental import pallas as pl
from jax.experimental.pallas import tpu as pltpu
from jax.experimental.pallas import tpu_sc as plsc

N_NODES = 10000
N_EDGES = 320000
D_FEAT = 128
HID = 128
N_CLASSES = 16

NC = 2
NS = 16
NW = NC * NS
LANES = 16
CH = 128
K_CHUNKS = -(-(-(-N_EDGES // (NW * CH))) // 4) * 4
KH = K_CHUNKS // 2
E_PER_TILE = K_CHUNKS * CH
E_PAD = NW * E_PER_TILE
ROWS_MAIN = (N_NODES // NS) // 8 * 8
ROWS_TAIL = N_NODES - ROWS_MAIN * NS

_PERM = np.arange(HID).reshape(HID // 32, 2, 16).transpose(0, 2, 1).reshape(-1)


def _seg_sum_sc(y, src, dst, w, d):
    bf16_in = y.dtype == jnp.bfloat16
    mesh = plsc.VectorSubcoreMesh(core_axis_name="c", subcore_axis_name="s")
    cp = pltpu.CompilerParams()
    if "needs_layout_passes" in pltpu.CompilerParams.__dataclass_fields__:
        cp = dataclasses.replace(cp, needs_layout_passes=False)
    cp = dataclasses.replace(cp, use_tc_tiling_on_sc=False)

    scratch = [
        pltpu.VMEM((KH, CH), jnp.int32),
        pltpu.VMEM((KH, CH), jnp.int32),
        pltpu.VMEM((KH, CH), jnp.float32),
        pltpu.VMEM((CH, d), y.dtype),
        pltpu.VMEM((CH, d), y.dtype),
        pltpu.VMEM((CH, d), jnp.float32),
        pltpu.VMEM_SHARED((N_NODES, d), jnp.float32),
        pltpu.SemaphoreType.DMA,
        pltpu.SemaphoreType.DMA,
    ]
    if not bf16_in:
        del scratch[5]

    @functools.partial(
        pl.kernel,
        mesh=mesh,
        compiler_params=cp,
        out_type=jax.ShapeDtypeStruct((NC, N_NODES, d), jnp.float32),
        scratch_types=scratch,
    )
    def seg_kernel(y_hbm, src_hbm, dst_hbm, w_hbm, out_hbm,
                   src_v, dst_v, w_v, rows0_v, rows1_v, *rest):
        if bf16_in:
            rowsf_v, acc_sh, gs0, gs1 = rest
        else:
            acc_sh, gs0, gs1 = rest
            rowsf_v = None
        rows_v = rowsf_v if bf16_in else rows0_v
        c = lax.axis_index("c")
        s = lax.axis_index("s")
        wid = c * NS + s

        @pl.loop(0, CH)
        def _(r):
            for j in range(d // LANES):
                rows_v[r, pl.ds(j * LANES, LANES)] = jnp.zeros(
                    (LANES,), jnp.float32)

        base = s * ROWS_MAIN
        for off in range(0, ROWS_MAIN, CH):
            n = min(CH, ROWS_MAIN - off)
            pltpu.sync_copy(rows_v.at[pl.ds(0, n)],
                            acc_sh.at[pl.ds(base + off, n)])

        @pl.when(s == NS - 1)
        def _():
            pltpu.sync_copy(rows_v.at[pl.ds(0, ROWS_TAIL)],
                            acc_sh.at[pl.ds(NS * ROWS_MAIN, ROWS_TAIL)])

        plsc.subcore_barrier()

        def scale_rows(buf, k):
            @pl.loop(0, CH)
            def _(e):
                wspl = plsc.load_gather(
                    w_v,
                    [jnp.full((LANES,), k, jnp.int32),
                     jnp.full((LANES,), e, jnp.int32)])
                if bf16_in:
                    for j in range(d // 32):
                        lo, hi = plsc.unpack(
                            buf[e, pl.ds(j * 32, 32)],
                            format=plsc.PackFormat.INTERLEAVED)
                        rowsf_v[e, pl.ds(j * 32, LANES)] = lo * wspl
                        rowsf_v[e, pl.ds(j * 32 + LANES, LANES)] = hi * wspl
                else:
                    for j in range(d // LANES):
                        sl = pl.ds(j * LANES, LANES)
                        buf[e, sl] = buf[e, sl] * wspl

        for h in range(2):
            h0 = h * KH
            pltpu.sync_copy(src_hbm.at[wid, pl.ds(h0, KH)], src_v)
            pltpu.sync_copy(dst_hbm.at[wid, pl.ds(h0, KH)], dst_v)
            pltpu.sync_copy(w_hbm.at[wid, pl.ds(h0, KH)], w_v)

            pltpu.async_copy(y_hbm.at[src_v.at[0]], rows0_v, gs0)
            pltpu.async_copy(y_hbm.at[src_v.at[1]], rows1_v, gs1)

            @pl.loop(0, KH, step=2)
            def _(k):
                for b, (buf, gs) in enumerate(((rows0_v, gs0),
                                               (rows1_v, gs1))):
                    kb = k + b
                    pltpu.make_async_copy(
                        y_hbm.at[src_v.at[kb]], buf, gs).wait()
                    scale_rows(buf, kb)
                    sbuf = rowsf_v if bf16_in else buf
                    pltpu.sync_copy(sbuf, acc_sh.at[dst_v.at[kb]], add=True)

                    @pl.when(kb + 2 < KH)
                    def _():
                        pltpu.async_copy(y_hbm.at[src_v.at[kb + 2]], buf, gs)

        plsc.subcore_barrier()

        pltpu.sync_copy(
            acc_sh.at[pl.ds(base, ROWS_MAIN)],
            out_hbm.at[c, pl.ds(base, ROWS_MAIN)])

        @pl.when(s == NS - 1)
        def _():
            pltpu.sync_copy(
                acc_sh.at[pl.ds(NS * ROWS_MAIN, ROWS_TAIL)],
                out_hbm.at[c, pl.ds(NS * ROWS_MAIN, ROWS_TAIL)])

    return seg_kernel(y, src, dst, w)


def _dot(a, b):
    return lax.dot_general(a, b, (((1,), (0,)), ((), ())),
                           precision=lax.Precision.HIGHEST,
                           preferred_element_type=jnp.float32)


_BR = 2000


def _proj1(x, w_rel, w_root):
    def body(x_ref, wa_ref, wb_ref, y_ref, r_ref):
        xv = x_ref[...]
        y_ref[...] = _dot(xv, wa_ref[...]).astype(jnp.bfloat16)
        r_ref[...] = _dot(xv, wb_ref[...])

    return pl.pallas_call(
        body,
        grid=(N_NODES // _BR,),
        in_specs=[
            pl.BlockSpec((_BR, D_FEAT), lambda i: (i, 0)),
            pl.BlockSpec((D_FEAT, HID), lambda i: (0, 0)),
            pl.BlockSpec((D_FEAT, HID), lambda i: (0, 0)),
        ],
        out_specs=[
            pl.BlockSpec((_BR, HID), lambda i: (i, 0)),
            pl.BlockSpec((_BR, HID), lambda i: (i, 0)),
        ],
        out_shape=[
            jax.ShapeDtypeStruct((N_NODES, HID), jnp.bfloat16),
            jax.ShapeDtypeStruct((N_NODES, HID), jnp.float32),
        ],
    )(x, w_rel, w_root)


def _layer2_proj(agg_p, r1, b1, w2_rel, w2_root, b2):
    def body(agg_ref, r1_ref, b1_ref, wa_ref, wb_ref, b2_ref, y2_ref, r2_ref):
        h = agg_ref[0] + agg_ref[1] + r1_ref[...] + b1_ref[...]
        h = jnp.maximum(h, 0.0)
        y2_ref[...] = _dot(h, wa_ref[...])
        r2_ref[...] = _dot(h, wb_ref[...]) + b2_ref[...]

    return pl.pallas_call(
        body,
        grid=(N_NODES // _BR,),
        in_specs=[
            pl.BlockSpec((NC, _BR, HID), lambda i: (0, i, 0)),
            pl.BlockSpec((_BR, HID), lambda i: (i, 0)),
            pl.BlockSpec((1, HID), lambda i: (0, 0)),
            pl.BlockSpec((HID, N_CLASSES), lambda i: (0, 0)),
            pl.BlockSpec((HID, N_CLASSES), lambda i: (0, 0)),
            pl.BlockSpec((1, N_CLASSES), lambda i: (0, 0)),
        ],
        out_specs=[
            pl.BlockSpec((_BR, N_CLASSES), lambda i: (i, 0)),
            pl.BlockSpec((_BR, N_CLASSES), lambda i: (i, 0)),
        ],
        out_shape=[
            jax.ShapeDtypeStruct((N_NODES, N_CLASSES), jnp.float32),
            jax.ShapeDtypeStruct((N_NODES, N_CLASSES), jnp.float32),
        ],
    )(agg_p, r1, b1, w2_rel, w2_root, b2)


def _final_sum(agg_p, r2b2):
    def body(agg_ref, r_ref, o_ref):
        o_ref[...] = agg_ref[0] + agg_ref[1] + r_ref[...]

    return pl.pallas_call(
        body,
        grid=(N_NODES // _BR,),
        in_specs=[
            pl.BlockSpec((NC, _BR, N_CLASSES), lambda i: (0, i, 0)),
            pl.BlockSpec((_BR, N_CLASSES), lambda i: (i, 0)),
        ],
        out_specs=pl.BlockSpec((_BR, N_CLASSES), lambda i: (i, 0)),
        out_shape=jax.ShapeDtypeStruct((N_NODES, N_CLASSES), jnp.float32),
    )(agg_p, r2b2)


def kernel(x, edge_index, edge_attr, W1_rel, b1_rel, W1_root,
           W2_rel, b2_rel, W2_root):
    src = edge_index[0].astype(jnp.int32)
    dst = edge_index[1].astype(jnp.int32)
    pad = E_PAD - N_EDGES
    src = jnp.pad(src, (0, pad)).reshape(NW, K_CHUNKS, CH)
    dst = jnp.pad(dst, (0, pad)).reshape(NW, K_CHUNKS, CH)
    w = jnp.pad(edge_attr, (0, pad)).reshape(NW, K_CHUNKS, CH)

    y1, r1 = _proj1(x, W1_rel[:, _PERM], W1_root)
    agg1 = _seg_sum_sc(y1, src, dst, w, HID)
    y2, r2b2 = _layer2_proj(agg1, r1, b1_rel.reshape(1, HID),
                            W2_rel, W2_root, b2_rel.reshape(1, N_CLASSES))
    agg2 = _seg_sum_sc(y2, src, dst, w, N_CLASSES)
    return _final_sum(agg2, r2b2)


# speedup vs baseline: 1.4117x; 1.0010x over previous
import dataclasses
import functools

import jax
import jax.numpy as jnp
import numpy as np
from jax import lax
from jax.experimental import pallas as pl
from jax.experimental.pallas import tpu as pltpu
from jax.experimental.pallas import tpu_sc as plsc

N_NODES = 10000
N_EDGES = 320000
D_FEAT = 128
HID = 128
N_CLASSES = 16

NC = 2
NS = 16
NW = NC * NS
LANES = 16
CH = 128
K_CHUNKS = -(-(-(-N_EDGES // (NW * CH))) // 4) * 4
KH = K_CHUNKS // 2
E_PER_TILE = K_CHUNKS * CH
E_PAD = NW * E_PER_TILE
ROWS_MAIN = (N_NODES // NS) // 8 * 8
ROWS_TAIL = N_NODES - ROWS_MAIN * NS

_PERM = np.arange(HID).reshape(HID // 32, 2, 16).transpose(0, 2, 1).reshape(-1)


def _seg_sum_sc(y, src, dst, w, d, g):
    bf16_in = y.dtype == jnp.bfloat16
    kb = K_CHUNKS // g
    kh = kb // 2
    rows = g * CH
    src = src.reshape(NW, kb, rows)
    dst = dst.reshape(NW, kb, g, CH)
    w = w.reshape(NW, kb, rows)
    mesh = plsc.VectorSubcoreMesh(core_axis_name="c", subcore_axis_name="s")
    cp = pltpu.CompilerParams()
    if "needs_layout_passes" in pltpu.CompilerParams.__dataclass_fields__:
        cp = dataclasses.replace(cp, needs_layout_passes=False)
    cp = dataclasses.replace(cp, use_tc_tiling_on_sc=False)

    scratch = [
        pltpu.VMEM((kh, rows), jnp.int32),
        pltpu.VMEM((kh, g, CH), jnp.int32),
        pltpu.VMEM((kh, rows), jnp.float32),
        pltpu.VMEM((rows, d), y.dtype),
        pltpu.VMEM((rows, d), y.dtype),
        pltpu.VMEM((rows, d), jnp.float32),
        pltpu.VMEM_SHARED((N_NODES, d), jnp.float32),
        pltpu.SemaphoreType.DMA,
        pltpu.SemaphoreType.DMA,
    ]
    if not bf16_in:
        del scratch[5]

    @functools.partial(
        pl.kernel,
        mesh=mesh,
        compiler_params=cp,
        out_type=jax.ShapeDtypeStruct((NC, N_NODES, d), jnp.float32),
        scratch_types=scratch,
    )
    def seg_kernel(y_hbm, src_hbm, dst_hbm, w_hbm, out_hbm,
                   src_v, dst_v, w_v, rows0_v, rows1_v, *rest):
        if bf16_in:
            rowsf_v, acc_sh, gs0, gs1 = rest
        else:
            acc_sh, gs0, gs1 = rest
            rowsf_v = None
        rows_v = rowsf_v if bf16_in else rows0_v
        c = lax.axis_index("c")
        s = lax.axis_index("s")
        wid = c * NS + s

        @pl.loop(0, rows)
        def _(r):
            for j in range(d // LANES):
                rows_v[r, pl.ds(j * LANES, LANES)] = jnp.zeros(
                    (LANES,), jnp.float32)

        base = s * ROWS_MAIN
        for off in range(0, ROWS_MAIN, rows):
            n = min(rows, ROWS_MAIN - off)
            pltpu.sync_copy(rows_v.at[pl.ds(0, n)],
                            acc_sh.at[pl.ds(base + off, n)])

        @pl.when(s == NS - 1)
        def _():
            pltpu.sync_copy(rows_v.at[pl.ds(0, ROWS_TAIL)],
                            acc_sh.at[pl.ds(NS * ROWS_MAIN, ROWS_TAIL)])

        plsc.subcore_barrier()

        def scale_rows(buf, k):
            @pl.loop(0, rows)
            def _(e):
                wspl = plsc.load_gather(
                    w_v,
                    [jnp.full((LANES,), k, jnp.int32),
                     jnp.full((LANES,), e, jnp.int32)])
                if bf16_in:
                    for j in range(d // 32):
                        lo, hi = plsc.unpack(
                            buf[e, pl.ds(j * 32, 32)],
                            format=plsc.PackFormat.INTERLEAVED)
                        rowsf_v[e, pl.ds(j * 32, LANES)] = lo * wspl
                        rowsf_v[e, pl.ds(j * 32 + LANES, LANES)] = hi * wspl
                else:
                    for j in range(d // LANES):
                        sl = pl.ds(j * LANES, LANES)
                        buf[e, sl] = buf[e, sl] * wspl

        for h in range(2):
            h0 = h * kh
            pltpu.sync_copy(src_hbm.at[wid, pl.ds(h0, kh)], src_v)
            pltpu.sync_copy(dst_hbm.at[wid, pl.ds(h0, kh)], dst_v)
            pltpu.sync_copy(w_hbm.at[wid, pl.ds(h0, kh)], w_v)

            pltpu.async_copy(y_hbm.at[src_v.at[0]], rows0_v, gs0)
            pltpu.async_copy(y_hbm.at[src_v.at[1]], rows1_v, gs1)

            @pl.loop(0, kh, step=2)
            def _(k):
                for b, (buf, gs) in enumerate(((rows0_v, gs0),
                                               (rows1_v, gs1))):
                    kk = k + b
                    pltpu.make_async_copy(
                        y_hbm.at[src_v.at[kk]], buf, gs).wait()
                    scale_rows(buf, kk)
                    sbuf = rowsf_v if bf16_in else buf
                    for j in range(g):
                        pltpu.sync_copy(sbuf.at[pl.ds(j * CH, CH)],
                                        acc_sh.at[dst_v.at[kk, j]],
                                        add=True)

                    @pl.when(kk + 2 < kh)
                    def _():
                        pltpu.async_copy(y_hbm.at[src_v.at[kk + 2]], buf, gs)

        plsc.subcore_barrier()

        pltpu.sync_copy(
            acc_sh.at[pl.ds(base, ROWS_MAIN)],
            out_hbm.at[c, pl.ds(base, ROWS_MAIN)])

        @pl.when(s == NS - 1)
        def _():
            pltpu.sync_copy(
                acc_sh.at[pl.ds(NS * ROWS_MAIN, ROWS_TAIL)],
                out_hbm.at[c, pl.ds(NS * ROWS_MAIN, ROWS_TAIL)])

    return seg_kernel(y, src, dst, w)


def _dot(a, b):
    return lax.dot_general(a, b, (((1,), (0,)), ((), ())),
                           precision=lax.Precision.HIGHEST,
                           preferred_element_type=jnp.float32)


_BR = 2000


def _proj1(x, w_rel, w_root):
    def body(x_ref, wa_ref, wb_ref, y_ref, r_ref):
        xv = x_ref[...]
        y_ref[...] = _dot(xv, wa_ref[...]).astype(jnp.bfloat16)
        r_ref[...] = _dot(xv, wb_ref[...])

    return pl.pallas_call(
        body,
        grid=(N_NODES // _BR,),
        in_specs=[
            pl.BlockSpec((_BR, D_FEAT), lambda i: (i, 0)),
            pl.BlockSpec((D_FEAT, HID), lambda i: (0, 0)),
            pl.BlockSpec((D_FEAT, HID), lambda i: (0, 0)),
        ],
        out_specs=[
            pl.BlockSpec((_BR, HID), lambda i: (i, 0)),
            pl.BlockSpec((_BR, HID), lambda i: (i, 0)),
        ],
        out_shape=[
            jax.ShapeDtypeStruct((N_NODES, HID), jnp.bfloat16),
            jax.ShapeDtypeStruct((N_NODES, HID), jnp.float32),
        ],
    )(x, w_rel, w_root)


def _layer2_proj(agg_p, r1, b1, w2_rel, w2_root, b2):
    def body(agg_ref, r1_ref, b1_ref, wa_ref, wb_ref, b2_ref, y2_ref, r2_ref):
        h = agg_ref[0] + agg_ref[1] + r1_ref[...] + b1_ref[...]
        h = jnp.maximum(h, 0.0)
        y2_ref[...] = _dot(h, wa_ref[...])
        r2_ref[...] = _dot(h, wb_ref[...]) + b2_ref[...]

    return pl.pallas_call(
        body,
        grid=(N_NODES // _BR,),
        in_specs=[
            pl.BlockSpec((NC, _BR, HID), lambda i: (0, i, 0)),
            pl.BlockSpec((_BR, HID), lambda i: (i, 0)),
            pl.BlockSpec((1, HID), lambda i: (0, 0)),
            pl.BlockSpec((HID, N_CLASSES), lambda i: (0, 0)),
            pl.BlockSpec((HID, N_CLASSES), lambda i: (0, 0)),
            pl.BlockSpec((1, N_CLASSES), lambda i: (0, 0)),
        ],
        out_specs=[
            pl.BlockSpec((_BR, N_CLASSES), lambda i: (i, 0)),
            pl.BlockSpec((_BR, N_CLASSES), lambda i: (i, 0)),
        ],
        out_shape=[
            jax.ShapeDtypeStruct((N_NODES, N_CLASSES), jnp.float32),
            jax.ShapeDtypeStruct((N_NODES, N_CLASSES), jnp.float32),
        ],
    )(agg_p, r1, b1, w2_rel, w2_root, b2)


def _final_sum(agg_p, r2b2):
    def body(agg_ref, r_ref, o_ref):
        o_ref[...] = agg_ref[0] + agg_ref[1] + r_ref[...]

    return pl.pallas_call(
        body,
        grid=(N_NODES // _BR,),
        in_specs=[
            pl.BlockSpec((NC, _BR, N_CLASSES), lambda i: (0, i, 0)),
            pl.BlockSpec((_BR, N_CLASSES), lambda i: (i, 0)),
        ],
        out_specs=pl.BlockSpec((_BR, N_CLASSES), lambda i: (i, 0)),
        out_shape=jax.ShapeDtypeStruct((N_NODES, N_CLASSES), jnp.float32),
    )(agg_p, r2b2)


def kernel(x, edge_index, edge_attr, W1_rel, b1_rel, W1_root,
           W2_rel, b2_rel, W2_root):
    src = edge_index[0].astype(jnp.int32)
    dst = edge_index[1].astype(jnp.int32)
    pad = E_PAD - N_EDGES
    src = jnp.pad(src, (0, pad)).reshape(NW, K_CHUNKS, CH)
    dst = jnp.pad(dst, (0, pad)).reshape(NW, K_CHUNKS, CH)
    w = jnp.pad(edge_attr, (0, pad)).reshape(NW, K_CHUNKS, CH)

    y1, r1 = _proj1(x, W1_rel[:, _PERM], W1_root)
    agg1 = _seg_sum_sc(y1, src, dst, w, HID, 1)
    y2, r2b2 = _layer2_proj(agg1, r1, b1_rel.reshape(1, HID),
                            W2_rel, W2_root, b2_rel.reshape(1, N_CLASSES))
    agg2 = _seg_sum_sc(y2, src, dst, w, N_CLASSES, 4)
    return _final_sum(agg2, r2b2)


# speedup vs baseline: 1.4163x; 1.0032x over previous
import dataclasses
import functools

import jax
import jax.numpy as jnp
import numpy as np
from jax import lax
from jax.experimental import pallas as pl
from jax.experimental.pallas import tpu as pltpu
from jax.experimental.pallas import tpu_sc as plsc

N_NODES = 10000
N_EDGES = 320000
D_FEAT = 128
HID = 128
N_CLASSES = 16

NC = 2
NS = 16
NW = NC * NS
LANES = 16
CH = 128
K_CHUNKS = -(-(-(-N_EDGES // (NW * CH))) // 4) * 4
KH = K_CHUNKS // 2
E_PER_TILE = K_CHUNKS * CH
E_PAD = NW * E_PER_TILE
ROWS_MAIN = (N_NODES // NS) // 8 * 8
ROWS_TAIL = N_NODES - ROWS_MAIN * NS

_PERM = np.arange(HID).reshape(HID // 32, 2, 16).transpose(0, 2, 1).reshape(-1)


def _seg_sum_sc(y, src, dst, w, d, g):
    bf16_in = y.dtype == jnp.bfloat16
    kb = K_CHUNKS // g
    kh = kb // 2
    rows = g * CH
    src = src.reshape(NW, kb, rows)
    dst = dst.reshape(NW, kb, g, CH)
    w = w.reshape(NW, kb, rows)
    mesh = plsc.VectorSubcoreMesh(core_axis_name="c", subcore_axis_name="s")
    cp = pltpu.CompilerParams()
    if "needs_layout_passes" in pltpu.CompilerParams.__dataclass_fields__:
        cp = dataclasses.replace(cp, needs_layout_passes=False)
    cp = dataclasses.replace(cp, use_tc_tiling_on_sc=False)

    scratch = [
        pltpu.VMEM((kh, rows), jnp.int32),
        pltpu.VMEM((kh, g, CH), jnp.int32),
        pltpu.VMEM((kh, rows), jnp.float32),
        pltpu.VMEM((rows, d), y.dtype),
        pltpu.VMEM((rows, d), y.dtype),
        pltpu.VMEM((rows, d), jnp.float32),
        pltpu.VMEM_SHARED((N_NODES, d), jnp.float32),
        pltpu.SemaphoreType.DMA,
        pltpu.SemaphoreType.DMA,
    ]

    @functools.partial(
        pl.kernel,
        mesh=mesh,
        compiler_params=cp,
        out_type=jax.ShapeDtypeStruct((NC, N_NODES, d), jnp.float32),
        scratch_types=scratch,
    )
    def seg_kernel(y_hbm, src_hbm, dst_hbm, w_hbm, out_hbm,
                   src_v, dst_v, w_v, rows0_v, rows1_v,
                   rowsf_v, acc_sh, gs0, gs1):
        c = lax.axis_index("c")
        s = lax.axis_index("s")
        wid = c * NS + s
        base = s * ROWS_MAIN

        def stage_half(h0):
            pltpu.sync_copy(src_hbm.at[wid, pl.ds(h0, kh)], src_v)
            pltpu.sync_copy(dst_hbm.at[wid, pl.ds(h0, kh)], dst_v)
            pltpu.sync_copy(w_hbm.at[wid, pl.ds(h0, kh)], w_v)

        stage_half(0)
        pltpu.async_copy(y_hbm.at[src_v.at[0]], rows0_v, gs0)
        pltpu.async_copy(y_hbm.at[src_v.at[1]], rows1_v, gs1)

        @pl.loop(0, rows)
        def _(r):
            for j in range(d // LANES):
                rowsf_v[r, pl.ds(j * LANES, LANES)] = jnp.zeros(
                    (LANES,), jnp.float32)

        for off in range(0, ROWS_MAIN, rows):
            n = min(rows, ROWS_MAIN - off)
            pltpu.sync_copy(rowsf_v.at[pl.ds(0, n)],
                            acc_sh.at[pl.ds(base + off, n)])

        @pl.when(s == NS - 1)
        def _():
            pltpu.sync_copy(rowsf_v.at[pl.ds(0, ROWS_TAIL)],
                            acc_sh.at[pl.ds(NS * ROWS_MAIN, ROWS_TAIL)])

        plsc.subcore_barrier()

        def scale_rows(buf, k):
            @pl.loop(0, rows)
            def _(e):
                wspl = plsc.load_gather(
                    w_v,
                    [jnp.full((LANES,), k, jnp.int32),
                     jnp.full((LANES,), e, jnp.int32)])
                if bf16_in:
                    for j in range(d // 32):
                        lo, hi = plsc.unpack(
                            buf[e, pl.ds(j * 32, 32)],
                            format=plsc.PackFormat.INTERLEAVED)
                        rowsf_v[e, pl.ds(j * 32, LANES)] = lo * wspl
                        rowsf_v[e, pl.ds(j * 32 + LANES, LANES)] = hi * wspl
                else:
                    for j in range(d // LANES):
                        sl = pl.ds(j * LANES, LANES)
                        buf[e, sl] = buf[e, sl] * wspl

        for h in range(2):
            if h:
                stage_half(h * kh)
                pltpu.async_copy(y_hbm.at[src_v.at[0]], rows0_v, gs0)
                pltpu.async_copy(y_hbm.at[src_v.at[1]], rows1_v, gs1)

            @pl.loop(0, kh, step=2)
            def _(k):
                for b, (buf, gs) in enumerate(((rows0_v, gs0),
                                               (rows1_v, gs1))):
                    kk = k + b
                    pltpu.make_async_copy(
                        y_hbm.at[src_v.at[kk]], buf, gs).wait()
                    scale_rows(buf, kk)
                    sbuf = rowsf_v if bf16_in else buf
                    for j in range(g):
                        pltpu.sync_copy(sbuf.at[pl.ds(j * CH, CH)],
                                        acc_sh.at[dst_v.at[kk, j]],
                                        add=True)

                    @pl.when(kk + 2 < kh)
                    def _():
                        pltpu.async_copy(y_hbm.at[src_v.at[kk + 2]], buf, gs)

        plsc.subcore_barrier()

        pltpu.sync_copy(
            acc_sh.at[pl.ds(base, ROWS_MAIN)],
            out_hbm.at[c, pl.ds(base, ROWS_MAIN)])

        @pl.when(s == NS - 1)
        def _():
            pltpu.sync_copy(
                acc_sh.at[pl.ds(NS * ROWS_MAIN, ROWS_TAIL)],
                out_hbm.at[c, pl.ds(NS * ROWS_MAIN, ROWS_TAIL)])

    return seg_kernel(y, src, dst, w)


def _dot(a, b):
    return lax.dot_general(a, b, (((1,), (0,)), ((), ())),
                           precision=lax.Precision.HIGHEST,
                           preferred_element_type=jnp.float32)


_BR = 2000


def _proj1(x, w_rel, w_root):
    def body(x_ref, wa_ref, wb_ref, y_ref, r_ref):
        xv = x_ref[...]
        y_ref[...] = _dot(xv, wa_ref[...]).astype(jnp.bfloat16)
        r_ref[...] = _dot(xv, wb_ref[...])

    return pl.pallas_call(
        body,
        grid=(N_NODES // _BR,),
        in_specs=[
            pl.BlockSpec((_BR, D_FEAT), lambda i: (i, 0)),
            pl.BlockSpec((D_FEAT, HID), lambda i: (0, 0)),
            pl.BlockSpec((D_FEAT, HID), lambda i: (0, 0)),
        ],
        out_specs=[
            pl.BlockSpec((_BR, HID), lambda i: (i, 0)),
            pl.BlockSpec((_BR, HID), lambda i: (i, 0)),
        ],
        out_shape=[
            jax.ShapeDtypeStruct((N_NODES, HID), jnp.bfloat16),
            jax.ShapeDtypeStruct((N_NODES, HID), jnp.float32),
        ],
    )(x, w_rel, w_root)


def _layer2_proj(agg_p, r1, b1, w2_rel, w2_root, b2):
    def body(agg_ref, r1_ref, b1_ref, wa_ref, wb_ref, b2_ref, y2_ref, r2_ref):
        h = agg_ref[0] + agg_ref[1] + r1_ref[...] + b1_ref[...]
        h = jnp.maximum(h, 0.0)
        y2_ref[...] = _dot(h, wa_ref[...])
        r2_ref[...] = _dot(h, wb_ref[...]) + b2_ref[...]

    return pl.pallas_call(
        body,
        grid=(N_NODES // _BR,),
        in_specs=[
            pl.BlockSpec((NC, _BR, HID), lambda i: (0, i, 0)),
            pl.BlockSpec((_BR, HID), lambda i: (i, 0)),
            pl.BlockSpec((1, HID), lambda i: (0, 0)),
            pl.BlockSpec((HID, N_CLASSES), lambda i: (0, 0)),
            pl.BlockSpec((HID, N_CLASSES), lambda i: (0, 0)),
            pl.BlockSpec((1, N_CLASSES), lambda i: (0, 0)),
        ],
        out_specs=[
            pl.BlockSpec((_BR, N_CLASSES), lambda i: (i, 0)),
            pl.BlockSpec((_BR, N_CLASSES), lambda i: (i, 0)),
        ],
        out_shape=[
            jax.ShapeDtypeStruct((N_NODES, N_CLASSES), jnp.float32),
            jax.ShapeDtypeStruct((N_NODES, N_CLASSES), jnp.float32),
        ],
    )(agg_p, r1, b1, w2_rel, w2_root, b2)


def _final_sum(agg_p, r2b2):
    def body(agg_ref, r_ref, o_ref):
        o_ref[...] = agg_ref[0] + agg_ref[1] + r_ref[...]

    return pl.pallas_call(
        body,
        grid=(N_NODES // _BR,),
        in_specs=[
            pl.BlockSpec((NC, _BR, N_CLASSES), lambda i: (0, i, 0)),
            pl.BlockSpec((_BR, N_CLASSES), lambda i: (i, 0)),
        ],
        out_specs=pl.BlockSpec((_BR, N_CLASSES), lambda i: (i, 0)),
        out_shape=jax.ShapeDtypeStruct((N_NODES, N_CLASSES), jnp.float32),
    )(agg_p, r2b2)


def kernel(x, edge_index, edge_attr, W1_rel, b1_rel, W1_root,
           W2_rel, b2_rel, W2_root):
    src = edge_index[0].astype(jnp.int32)
    dst = edge_index[1].astype(jnp.int32)
    pad = E_PAD - N_EDGES
    src = jnp.pad(src, (0, pad)).reshape(NW, K_CHUNKS, CH)
    dst = jnp.pad(dst, (0, pad)).reshape(NW, K_CHUNKS, CH)
    w = jnp.pad(edge_attr, (0, pad)).reshape(NW, K_CHUNKS, CH)

    y1, r1 = _proj1(x, W1_rel[:, _PERM], W1_root)
    agg1 = _seg_sum_sc(y1, src, dst, w, HID, 1)
    y2, r2b2 = _layer2_proj(agg1, r1, b1_rel.reshape(1, HID),
                            W2_rel, W2_root, b2_rel.reshape(1, N_CLASSES))
    agg2 = _seg_sum_sc(y2, src, dst, w, N_CLASSES, 4)
    return _final_sum(agg2, r2b2)


# speedup vs baseline: 1.7106x; 1.2079x over previous
import dataclasses
import functools

import jax
import jax.numpy as jnp
import numpy as np
from jax import lax
from jax.experimental import pallas as pl
from jax.experimental.pallas import tpu as pltpu
from jax.experimental.pallas import tpu_sc as plsc

N_NODES = 10000
N_EDGES = 320000
D_FEAT = 128
HID = 128
N_CLASSES = 16

NC = 2
NS = 16
NW = NC * NS
LANES = 16
CH = 128
K_CHUNKS = -(-(-(-N_EDGES // (NW * CH))) // 4) * 4
KH = K_CHUNKS // 2
E_PER_TILE = K_CHUNKS * CH
E_PAD = NW * E_PER_TILE
ROWS_MAIN = (N_NODES // NS) // 8 * 8
ROWS_TAIL = N_NODES - ROWS_MAIN * NS

_PERM = np.arange(HID).reshape(HID // 32, 2, 16).transpose(0, 2, 1).reshape(-1)


def _seg_sum_sc(y, src, dst, w, d, g):
    bf16_in = y.dtype == jnp.bfloat16
    kb = K_CHUNKS // g
    kh = kb // 2
    rows = g * CH
    src = src.reshape(NW, kb, rows)
    dst = dst.reshape(NW, kb, g, CH)
    w = w.reshape(NW, kb, rows)
    mesh = plsc.VectorSubcoreMesh(core_axis_name="c", subcore_axis_name="s")
    cp = pltpu.CompilerParams()
    if "needs_layout_passes" in pltpu.CompilerParams.__dataclass_fields__:
        cp = dataclasses.replace(cp, needs_layout_passes=False)
    cp = dataclasses.replace(cp, use_tc_tiling_on_sc=False)

    scratch = [
        pltpu.VMEM((kh, rows), jnp.int32),
        pltpu.VMEM((kh, g, CH), jnp.int32),
        pltpu.VMEM((kh, rows), jnp.float32),
        pltpu.VMEM((rows, d), y.dtype),
        pltpu.VMEM((rows, d), y.dtype),
        pltpu.VMEM((rows, d), jnp.float32),
        pltpu.VMEM_SHARED((N_NODES, d), jnp.float32),
        pltpu.SemaphoreType.DMA,
        pltpu.SemaphoreType.DMA,
    ]

    @functools.partial(
        pl.kernel,
        mesh=mesh,
        compiler_params=cp,
        out_type=jax.ShapeDtypeStruct((NC, N_NODES, d), jnp.float32),
        scratch_types=scratch,
    )
    def seg_kernel(y_hbm, src_hbm, dst_hbm, w_hbm, out_hbm,
                   src_v, dst_v, w_v, rows0_v, rows1_v,
                   rowsf_v, acc_sh, gs0, gs1):
        c = lax.axis_index("c")
        s = lax.axis_index("s")
        wid = c * NS + s
        base = s * ROWS_MAIN

        def stage_half(h0):
            pltpu.sync_copy(src_hbm.at[wid, pl.ds(h0, kh)], src_v)
            pltpu.sync_copy(dst_hbm.at[wid, pl.ds(h0, kh)], dst_v)
            pltpu.sync_copy(w_hbm.at[wid, pl.ds(h0, kh)], w_v)

        stage_half(0)
        pltpu.async_copy(y_hbm.at[src_v.at[0]], rows0_v, gs0)
        pltpu.async_copy(y_hbm.at[src_v.at[1]], rows1_v, gs1)

        @pl.loop(0, rows)
        def _(r):
            for j in range(d // LANES):
                rowsf_v[r, pl.ds(j * LANES, LANES)] = jnp.zeros(
                    (LANES,), jnp.float32)

        for off in range(0, ROWS_MAIN, rows):
            n = min(rows, ROWS_MAIN - off)
            pltpu.sync_copy(rowsf_v.at[pl.ds(0, n)],
                            acc_sh.at[pl.ds(base + off, n)])

        @pl.when(s == NS - 1)
        def _():
            pltpu.sync_copy(rowsf_v.at[pl.ds(0, ROWS_TAIL)],
                            acc_sh.at[pl.ds(NS * ROWS_MAIN, ROWS_TAIL)])

        plsc.subcore_barrier()

        def scale_rows(buf, k):
            @plsc.parallel_loop(0, rows, unroll=2)
            def _(e):
                wspl = plsc.load_gather(
                    w_v,
                    [jnp.full((LANES,), k, jnp.int32),
                     jnp.full((LANES,), e, jnp.int32)])
                if bf16_in:
                    for j in range(d // 32):
                        lo, hi = plsc.unpack(
                            buf[e, pl.ds(j * 32, 32)],
                            format=plsc.PackFormat.INTERLEAVED)
                        rowsf_v[e, pl.ds(j * 32, LANES)] = lo * wspl
                        rowsf_v[e, pl.ds(j * 32 + LANES, LANES)] = hi * wspl
                else:
                    for j in range(d // LANES):
                        sl = pl.ds(j * LANES, LANES)
                        buf[e, sl] = buf[e, sl] * wspl

        for h in range(2):
            if h:
                stage_half(h * kh)
                pltpu.async_copy(y_hbm.at[src_v.at[0]], rows0_v, gs0)
                pltpu.async_copy(y_hbm.at[src_v.at[1]], rows1_v, gs1)

            @pl.loop(0, kh, step=2)
            def _(k):
                for b, (buf, gs) in enumerate(((rows0_v, gs0),
                                               (rows1_v, gs1))):
                    kk = k + b
                    pltpu.make_async_copy(
                        y_hbm.at[src_v.at[kk]], buf, gs).wait()
                    scale_rows(buf, kk)
                    sbuf = rowsf_v if bf16_in else buf
                    for j in range(g):
                        pltpu.sync_copy(sbuf.at[pl.ds(j * CH, CH)],
                                        acc_sh.at[dst_v.at[kk, j]],
                                        add=True)

                    @pl.when(kk + 2 < kh)
                    def _():
                        pltpu.async_copy(y_hbm.at[src_v.at[kk + 2]], buf, gs)

        plsc.subcore_barrier()

        pltpu.sync_copy(
            acc_sh.at[pl.ds(base, ROWS_MAIN)],
            out_hbm.at[c, pl.ds(base, ROWS_MAIN)])

        @pl.when(s == NS - 1)
        def _():
            pltpu.sync_copy(
                acc_sh.at[pl.ds(NS * ROWS_MAIN, ROWS_TAIL)],
                out_hbm.at[c, pl.ds(NS * ROWS_MAIN, ROWS_TAIL)])

    return seg_kernel(y, src, dst, w)


def _dot(a, b):
    return lax.dot_general(a, b, (((1,), (0,)), ((), ())),
                           precision=lax.Precision.HIGHEST,
                           preferred_element_type=jnp.float32)


_BR = 2000


def _proj1(x, w_rel, w_root):
    def body(x_ref, wa_ref, wb_ref, y_ref, r_ref):
        xv = x_ref[...]
        y_ref[...] = _dot(xv, wa_ref[...]).astype(jnp.bfloat16)
        r_ref[...] = _dot(xv, wb_ref[...])

    return pl.pallas_call(
        body,
        grid=(N_NODES // _BR,),
        in_specs=[
            pl.BlockSpec((_BR, D_FEAT), lambda i: (i, 0)),
            pl.BlockSpec((D_FEAT, HID), lambda i: (0, 0)),
            pl.BlockSpec((D_FEAT, HID), lambda i: (0, 0)),
        ],
        out_specs=[
            pl.BlockSpec((_BR, HID), lambda i: (i, 0)),
            pl.BlockSpec((_BR, HID), lambda i: (i, 0)),
        ],
        out_shape=[
            jax.ShapeDtypeStruct((N_NODES, HID), jnp.bfloat16),
            jax.ShapeDtypeStruct((N_NODES, HID), jnp.float32),
        ],
    )(x, w_rel, w_root)


def _layer2_proj(agg_p, r1, b1, w2_rel, w2_root, b2):
    def body(agg_ref, r1_ref, b1_ref, wa_ref, wb_ref, b2_ref, y2_ref, r2_ref):
        h = agg_ref[0] + agg_ref[1] + r1_ref[...] + b1_ref[...]
        h = jnp.maximum(h, 0.0)
        y2_ref[...] = _dot(h, wa_ref[...])
        r2_ref[...] = _dot(h, wb_ref[...]) + b2_ref[...]

    return pl.pallas_call(
        body,
        grid=(N_NODES // _BR,),
        in_specs=[
            pl.BlockSpec((NC, _BR, HID), lambda i: (0, i, 0)),
            pl.BlockSpec((_BR, HID), lambda i: (i, 0)),
            pl.BlockSpec((1, HID), lambda i: (0, 0)),
            pl.BlockSpec((HID, N_CLASSES), lambda i: (0, 0)),
            pl.BlockSpec((HID, N_CLASSES), lambda i: (0, 0)),
            pl.BlockSpec((1, N_CLASSES), lambda i: (0, 0)),
        ],
        out_specs=[
            pl.BlockSpec((_BR, N_CLASSES), lambda i: (i, 0)),
            pl.BlockSpec((_BR, N_CLASSES), lambda i: (i, 0)),
        ],
        out_shape=[
            jax.ShapeDtypeStruct((N_NODES, N_CLASSES), jnp.float32),
            jax.ShapeDtypeStruct((N_NODES, N_CLASSES), jnp.float32),
        ],
    )(agg_p, r1, b1, w2_rel, w2_root, b2)


def _final_sum(agg_p, r2b2):
    def body(agg_ref, r_ref, o_ref):
        o_ref[...] = agg_ref[0] + agg_ref[1] + r_ref[...]

    return pl.pallas_call(
        body,
        grid=(N_NODES // _BR,),
        in_specs=[
            pl.BlockSpec((NC, _BR, N_CLASSES), lambda i: (0, i, 0)),
            pl.BlockSpec((_BR, N_CLASSES), lambda i: (i, 0)),
        ],
        out_specs=pl.BlockSpec((_BR, N_CLASSES), lambda i: (i, 0)),
        out_shape=jax.ShapeDtypeStruct((N_NODES, N_CLASSES), jnp.float32),
    )(agg_p, r2b2)


def kernel(x, edge_index, edge_attr, W1_rel, b1_rel, W1_root,
           W2_rel, b2_rel, W2_root):
    src = edge_index[0].astype(jnp.int32)
    dst = edge_index[1].astype(jnp.int32)
    pad = E_PAD - N_EDGES
    src = jnp.pad(src, (0, pad)).reshape(NW, K_CHUNKS, CH)
    dst = jnp.pad(dst, (0, pad)).reshape(NW, K_CHUNKS, CH)
    w = jnp.pad(edge_attr, (0, pad)).reshape(NW, K_CHUNKS, CH)

    y1, r1 = _proj1(x, W1_rel[:, _PERM], W1_root)
    agg1 = _seg_sum_sc(y1, src, dst, w, HID, 1)
    y2, r2b2 = _layer2_proj(agg1, r1, b1_rel.reshape(1, HID),
                            W2_rel, W2_root, b2_rel.reshape(1, N_CLASSES))
    agg2 = _seg_sum_sc(y2, src, dst, w, N_CLASSES, 4)
    return _final_sum(agg2, r2b2)


# speedup vs baseline: 1.7363x; 1.0150x over previous
import dataclasses
import functools

import jax
import jax.numpy as jnp
import numpy as np
from jax import lax
from jax.experimental import pallas as pl
from jax.experimental.pallas import tpu as pltpu
from jax.experimental.pallas import tpu_sc as plsc

N_NODES = 10000
N_EDGES = 320000
D_FEAT = 128
HID = 128
N_CLASSES = 16

NC = 2
NS = 16
NW = NC * NS
LANES = 16
CH = 128
K_CHUNKS = -(-(-(-N_EDGES // (NW * CH))) // 4) * 4
KH = K_CHUNKS // 2
E_PER_TILE = K_CHUNKS * CH
E_PAD = NW * E_PER_TILE
ROWS_MAIN = (N_NODES // NS) // 8 * 8
ROWS_TAIL = N_NODES - ROWS_MAIN * NS

_PERM = np.arange(HID).reshape(HID // 32, 2, 16).transpose(0, 2, 1).reshape(-1)


def _seg_sum_sc(y, src, dst, w, d, g):
    bf16_in = y.dtype == jnp.bfloat16
    kb = K_CHUNKS // g
    kh = kb // 2
    rows = g * CH
    src = src.reshape(NW, kb, rows)
    dst = dst.reshape(NW, kb, g, CH)
    w = w.reshape(NW, kb, rows)
    mesh = plsc.VectorSubcoreMesh(core_axis_name="c", subcore_axis_name="s")
    cp = pltpu.CompilerParams()
    if "needs_layout_passes" in pltpu.CompilerParams.__dataclass_fields__:
        cp = dataclasses.replace(cp, needs_layout_passes=False)
    cp = dataclasses.replace(cp, use_tc_tiling_on_sc=False)

    scratch = [
        pltpu.VMEM((kh, rows), jnp.int32),
        pltpu.VMEM((kh, g, CH), jnp.int32),
        pltpu.VMEM((kh, rows), jnp.float32),
        pltpu.VMEM((rows, d), y.dtype),
        pltpu.VMEM((rows, d), y.dtype),
        pltpu.VMEM((rows, d), jnp.float32),
        pltpu.VMEM_SHARED((N_NODES, d), jnp.float32),
        pltpu.SemaphoreType.DMA,
        pltpu.SemaphoreType.DMA,
    ]

    @functools.partial(
        pl.kernel,
        mesh=mesh,
        compiler_params=cp,
        out_type=jax.ShapeDtypeStruct((NC, N_NODES, d), jnp.float32),
        scratch_types=scratch,
    )
    def seg_kernel(y_hbm, src_hbm, dst_hbm, w_hbm, out_hbm,
                   src_v, dst_v, w_v, rows0_v, rows1_v,
                   rowsf_v, acc_sh, gs0, gs1):
        c = lax.axis_index("c")
        s = lax.axis_index("s")
        wid = c * NS + s
        base = s * ROWS_MAIN

        def stage_half(h0):
            pltpu.sync_copy(src_hbm.at[wid, pl.ds(h0, kh)], src_v)
            pltpu.sync_copy(dst_hbm.at[wid, pl.ds(h0, kh)], dst_v)
            pltpu.sync_copy(w_hbm.at[wid, pl.ds(h0, kh)], w_v)

        stage_half(0)
        pltpu.async_copy(y_hbm.at[src_v.at[0]], rows0_v, gs0)
        pltpu.async_copy(y_hbm.at[src_v.at[1]], rows1_v, gs1)

        @pl.loop(0, rows)
        def _(r):
            for j in range(d // LANES):
                rowsf_v[r, pl.ds(j * LANES, LANES)] = jnp.zeros(
                    (LANES,), jnp.float32)

        for off in range(0, ROWS_MAIN, rows):
            n = min(rows, ROWS_MAIN - off)
            pltpu.sync_copy(rowsf_v.at[pl.ds(0, n)],
                            acc_sh.at[pl.ds(base + off, n)])

        @pl.when(s == NS - 1)
        def _():
            pltpu.sync_copy(rowsf_v.at[pl.ds(0, ROWS_TAIL)],
                            acc_sh.at[pl.ds(NS * ROWS_MAIN, ROWS_TAIL)])

        plsc.subcore_barrier()

        def scale_rows(buf, k):
            @plsc.parallel_loop(0, rows, unroll=4)
            def _(e):
                wspl = plsc.load_gather(
                    w_v,
                    [jnp.full((LANES,), k, jnp.int32),
                     jnp.full((LANES,), e, jnp.int32)])
                if bf16_in:
                    for j in range(d // 32):
                        lo, hi = plsc.unpack(
                            buf[e, pl.ds(j * 32, 32)],
                            format=plsc.PackFormat.INTERLEAVED)
                        rowsf_v[e, pl.ds(j * 32, LANES)] = lo * wspl
                        rowsf_v[e, pl.ds(j * 32 + LANES, LANES)] = hi * wspl
                else:
                    for j in range(d // LANES):
                        sl = pl.ds(j * LANES, LANES)
                        buf[e, sl] = buf[e, sl] * wspl

        for h in range(2):
            if h:
                stage_half(h * kh)
                pltpu.async_copy(y_hbm.at[src_v.at[0]], rows0_v, gs0)
                pltpu.async_copy(y_hbm.at[src_v.at[1]], rows1_v, gs1)

            @pl.loop(0, kh, step=2)
            def _(k):
                for b, (buf, gs) in enumerate(((rows0_v, gs0),
                                               (rows1_v, gs1))):
                    kk = k + b
                    pltpu.make_async_copy(
                        y_hbm.at[src_v.at[kk]], buf, gs).wait()
                    scale_rows(buf, kk)
                    sbuf = rowsf_v if bf16_in else buf
                    for j in range(g):
                        pltpu.sync_copy(sbuf.at[pl.ds(j * CH, CH)],
                                        acc_sh.at[dst_v.at[kk, j]],
                                        add=True)

                    @pl.when(kk + 2 < kh)
                    def _():
                        pltpu.async_copy(y_hbm.at[src_v.at[kk + 2]], buf, gs)

        plsc.subcore_barrier()

        pltpu.sync_copy(
            acc_sh.at[pl.ds(base, ROWS_MAIN)],
            out_hbm.at[c, pl.ds(base, ROWS_MAIN)])

        @pl.when(s == NS - 1)
        def _():
            pltpu.sync_copy(
                acc_sh.at[pl.ds(NS * ROWS_MAIN, ROWS_TAIL)],
                out_hbm.at[c, pl.ds(NS * ROWS_MAIN, ROWS_TAIL)])

    return seg_kernel(y, src, dst, w)


def _dot(a, b):
    return lax.dot_general(a, b, (((1,), (0,)), ((), ())),
                           precision=lax.Precision.HIGHEST,
                           preferred_element_type=jnp.float32)


_BR = 2000


def _proj1(x, w_rel, w_root):
    def body(x_ref, wa_ref, wb_ref, y_ref, r_ref):
        xv = x_ref[...]
        y_ref[...] = _dot(xv, wa_ref[...]).astype(jnp.bfloat16)
        r_ref[...] = _dot(xv, wb_ref[...])

    return pl.pallas_call(
        body,
        grid=(N_NODES // _BR,),
        in_specs=[
            pl.BlockSpec((_BR, D_FEAT), lambda i: (i, 0)),
            pl.BlockSpec((D_FEAT, HID), lambda i: (0, 0)),
            pl.BlockSpec((D_FEAT, HID), lambda i: (0, 0)),
        ],
        out_specs=[
            pl.BlockSpec((_BR, HID), lambda i: (i, 0)),
            pl.BlockSpec((_BR, HID), lambda i: (i, 0)),
        ],
        out_shape=[
            jax.ShapeDtypeStruct((N_NODES, HID), jnp.bfloat16),
            jax.ShapeDtypeStruct((N_NODES, HID), jnp.float32),
        ],
    )(x, w_rel, w_root)


def _layer2_proj(agg_p, r1, b1, w2_rel, w2_root, b2):
    def body(agg_ref, r1_ref, b1_ref, wa_ref, wb_ref, b2_ref, y2_ref, r2_ref):
        h = agg_ref[0] + agg_ref[1] + r1_ref[...] + b1_ref[...]
        h = jnp.maximum(h, 0.0)
        y2_ref[...] = _dot(h, wa_ref[...])
        r2_ref[...] = _dot(h, wb_ref[...]) + b2_ref[...]

    return pl.pallas_call(
        body,
        grid=(N_NODES // _BR,),
        in_specs=[
            pl.BlockSpec((NC, _BR, HID), lambda i: (0, i, 0)),
            pl.BlockSpec((_BR, HID), lambda i: (i, 0)),
            pl.BlockSpec((1, HID), lambda i: (0, 0)),
            pl.BlockSpec((HID, N_CLASSES), lambda i: (0, 0)),
            pl.BlockSpec((HID, N_CLASSES), lambda i: (0, 0)),
            pl.BlockSpec((1, N_CLASSES), lambda i: (0, 0)),
        ],
        out_specs=[
            pl.BlockSpec((_BR, N_CLASSES), lambda i: (i, 0)),
            pl.BlockSpec((_BR, N_CLASSES), lambda i: (i, 0)),
        ],
        out_shape=[
            jax.ShapeDtypeStruct((N_NODES, N_CLASSES), jnp.float32),
            jax.ShapeDtypeStruct((N_NODES, N_CLASSES), jnp.float32),
        ],
    )(agg_p, r1, b1, w2_rel, w2_root, b2)


def _final_sum(agg_p, r2b2):
    def body(agg_ref, r_ref, o_ref):
        o_ref[...] = agg_ref[0] + agg_ref[1] + r_ref[...]

    return pl.pallas_call(
        body,
        grid=(N_NODES // _BR,),
        in_specs=[
            pl.BlockSpec((NC, _BR, N_CLASSES), lambda i: (0, i, 0)),
            pl.BlockSpec((_BR, N_CLASSES), lambda i: (i, 0)),
        ],
        out_specs=pl.BlockSpec((_BR, N_CLASSES), lambda i: (i, 0)),
        out_shape=jax.ShapeDtypeStruct((N_NODES, N_CLASSES), jnp.float32),
    )(agg_p, r2b2)


def kernel(x, edge_index, edge_attr, W1_rel, b1_rel, W1_root,
           W2_rel, b2_rel, W2_root):
    src = edge_index[0].astype(jnp.int32)
    dst = edge_index[1].astype(jnp.int32)
    pad = E_PAD - N_EDGES
    src = jnp.pad(src, (0, pad)).reshape(NW, K_CHUNKS, CH)
    dst = jnp.pad(dst, (0, pad)).reshape(NW, K_CHUNKS, CH)
    w = jnp.pad(edge_attr, (0, pad)).reshape(NW, K_CHUNKS, CH)

    y1, r1 = _proj1(x, W1_rel[:, _PERM], W1_root)
    agg1 = _seg_sum_sc(y1, src, dst, w, HID, 1)
    y2, r2b2 = _layer2_proj(agg1, r1, b1_rel.reshape(1, HID),
                            W2_rel, W2_root, b2_rel.reshape(1, N_CLASSES))
    agg2 = _seg_sum_sc(y2, src, dst, w, N_CLASSES, 4)
    return _final_sum(agg2, r2b2)
